# Initial kernel scaffold; baseline (speedup 1.0000x reference)
import jax
import jax.numpy as jnp
from jax.experimental import pallas as pl


def kernel(x, W_stem, b_stem, pos_embed, W_e1, b_e1, W_e2, b_e2, W_msg, b_msg, W_f1, b_f1, W_f2, b_f2, W_p1, b_p1, gamma, beta, W_p2, b_p2):
    raise NotImplementedError("write your pallas kernel here")


# speedup vs baseline: 1.3214x; 1.3214x over previous
import jax
import jax.numpy as jnp
from jax import lax
from jax.experimental import pallas as pl
from jax.experimental.pallas import tpu as pltpu

B = 64
C = 192
P = 16
H = 14
W = 14
N = H * W
K = 5
PATCH = 3 * P * P


def _gsat_body(patches_ref, pos_ref, Ws_ref, bstem_ref, W1a_ref, W1b_ref,
               be1_ref, w2r_ref, be2_ref, Wm_ref, bm_ref, Wf1_ref, bf1_ref,
               Wf2_ref, bf2_ref, att_ref, gv_ref):
    f32 = jnp.float32
    patches = patches_ref[0]
    nodes = jnp.dot(patches, Ws_ref[...], preferred_element_type=f32)
    nodes = nodes + bstem_ref[...] + pos_ref[...]

    G = lax.dot_general(nodes, nodes, (((1,), (1,)), ((), ())),
                        preferred_element_type=f32)
    sq = jnp.sum(nodes * nodes, axis=1)
    ri = lax.broadcasted_iota(jnp.int32, (N, N), 0)
    ci = lax.broadcasted_iota(jnp.int32, (N, N), 1)
    scores = sq[None, :] - 2.0 * G
    scores = jnp.where(ri == ci, f32(1e10), scores)

    A = jnp.dot(nodes, W1a_ref[...], preferred_element_type=f32)
    Bm = jnp.dot(nodes, W1b_ref[...], preferred_element_type=f32) + be1_ref[...]

    agg = jnp.zeros((N, C), f32)
    atts = []
    for _ in range(K):
        m = jnp.min(scores, axis=1, keepdims=True)
        amin = jnp.min(jnp.where(scores == m, ci, N), axis=1, keepdims=True)
        sel = ci == amin
        onehot = jnp.where(sel, f32(1.0), f32(0.0))
        scores = jnp.where(sel, f32(1e30), scores)
        Asrc = jnp.dot(onehot, A, preferred_element_type=f32,
                       precision=lax.Precision.HIGHEST)
        Fsrc = jnp.dot(onehot, nodes, preferred_element_type=f32,
                       precision=lax.Precision.HIGHEST)
        e = jnp.maximum(Asrc + Bm, 0.0)
        logit = jnp.sum(e * w2r_ref[...], axis=1, keepdims=True) + be2_ref[...]
        att = 1.0 / (1.0 + jnp.exp(-logit))
        atts.append(att)
        agg = agg + att * Fsrc
    att_ref[0] = jnp.concatenate(atts, axis=1)

    ne = jnp.maximum(
        jnp.dot(nodes + agg, Wm_ref[...], preferred_element_type=f32)
        + bm_ref[...], 0.0)
    hidden = jax.nn.gelu(
        jnp.dot(ne, Wf1_ref[...], preferred_element_type=f32) + bf1_ref[...])
    nm = ne + jnp.dot(hidden, Wf2_ref[...], preferred_element_type=f32) \
        + bf2_ref[...]
    gv_ref[0, 0, :] = jnp.sum(nm, axis=0) * f32(1.0 / N)


def _head_body(g_ref, Wp1_ref, bp1_ref, gamma_ref, beta_ref, wp2r_ref,
               bp2_ref, out_ref):
    f32 = jnp.float32
    p = jnp.dot(g_ref[...], Wp1_ref[...], preferred_element_type=f32) \
        + bp1_ref[...]
    p = jax.nn.gelu(p * gamma_ref[...] + beta_ref[...])
    out_ref[...] = jnp.sum(p * wp2r_ref[...], axis=1, keepdims=True) \
        + bp2_ref[...]


def kernel(x, W_stem, b_stem, pos_embed, W_e1, b_e1, W_e2, b_e2, W_msg, b_msg,
           W_f1, b_f1, W_f2, b_f2, W_p1, b_p1, gamma, beta, W_p2, b_p2):
    f32 = jnp.float32
    patches = x.reshape(B, 3, H, P, W, P).transpose(0, 2, 4, 1, 3, 5) \
        .reshape(B, N, PATCH)
    Ws = W_stem.reshape(C, PATCH).T
    pos = pos_embed.transpose(0, 2, 3, 1).reshape(N, C)
    W1a = W_e1[:C]
    W1b = W_e1[C:]

    const2d = lambda: pl.BlockSpec(index_map=lambda i: (0, 0))
    att, gv = pl.pallas_call(
        _gsat_body,
        grid=(B,),
        in_specs=[
            pl.BlockSpec((1, N, PATCH), lambda i: (i, 0, 0)),
            const2d(),
            const2d(),
            const2d(),
            const2d(),
            const2d(),
            const2d(),
            const2d(),
            const2d(),
            const2d(),
            const2d(),
            const2d(),
            const2d(),
            const2d(),
            const2d(),
        ],
        out_specs=[
            pl.BlockSpec((1, N, K), lambda i: (i, 0, 0)),
            pl.BlockSpec((1, 1, C), lambda i: (i, 0, 0)),
        ],
        out_shape=[
            jax.ShapeDtypeStruct((B, N, K), f32),
            jax.ShapeDtypeStruct((B, 1, C), f32),
        ],
        compiler_params=pltpu.CompilerParams(
            dimension_semantics=("parallel",)),
    )(patches, pos, Ws, b_stem.reshape(1, C), W1a, W1b, b_e1.reshape(1, C),
      W_e2.reshape(1, C), b_e2.reshape(1, 1), W_msg, b_msg.reshape(1, C),
      W_f1, b_f1.reshape(1, 4 * C), W_f2, b_f2.reshape(1, C))

    pred = pl.pallas_call(
        _head_body,
        out_shape=jax.ShapeDtypeStruct((B, 1), f32),
    )(gv.reshape(B, C), W_p1, b_p1.reshape(1, 1024), gamma.reshape(1, 1024),
      beta.reshape(1, 1024), W_p2.reshape(1, 1024), b_p2.reshape(1, 1))

    return (att.reshape(B * N * K, 1), pred)


# speedup vs baseline: 3.7996x; 2.8755x over previous
import jax
import jax.numpy as jnp
from jax import lax
from jax.experimental import pallas as pl
from jax.experimental.pallas import tpu as pltpu

B = 64
C = 192
P = 16
H = 14
W = 14
N = H * W
K = 5
PATCH = 3 * P * P


def _gsat_body(patches_ref, pos_ref, Ws_ref, bstem_ref, W1a_ref, W1b_ref,
               be1_ref, w2c_ref, be2_ref, Wm_ref, bm_ref, Wf1_ref, bf1_ref,
               Wf2_ref, bf2_ref, att_ref, gv_ref):
    f32 = jnp.float32
    patches = patches_ref[0]
    nodes = jnp.dot(patches, Ws_ref[...], preferred_element_type=f32)
    nodes = nodes + bstem_ref[...] + pos_ref[...]

    G = lax.dot_general(nodes, nodes, (((1,), (1,)), ((), ())),
                        preferred_element_type=f32)
    nn = nodes * nodes
    sqrow = lax.dot_general(jnp.ones((1, C), f32), nn,
                            (((1,), (1,)), ((), ())),
                            preferred_element_type=f32,
                            precision=lax.Precision.HIGHEST)
    scores = sqrow - 2.0 * G
    ri = lax.broadcasted_iota(jnp.int32, (N, N), 0)
    ci = lax.broadcasted_iota(jnp.int32, (N, N), 1)
    scores = jnp.where(ri == ci, f32(1e10), scores)

    A = jnp.dot(nodes, W1a_ref[...], preferred_element_type=f32)
    Bm = jnp.dot(nodes, W1b_ref[...], preferred_element_type=f32) + be1_ref[...]

    agg = jnp.zeros((N, C), f32)
    atts = []
    for _ in range(K):
        m = jnp.min(scores, axis=1, keepdims=True)
        amin = jnp.min(jnp.where(scores == m, ci, N), axis=1, keepdims=True)
        sel = ci == amin
        onehot = jnp.where(sel, f32(1.0), f32(0.0))
        scores = jnp.where(sel, f32(1e30), scores)
        Asrc = jnp.dot(onehot, A, preferred_element_type=f32)
        Fsrc = jnp.dot(onehot, nodes, preferred_element_type=f32)
        e = jnp.maximum(Asrc + Bm, 0.0)
        logit = jnp.dot(e, w2c_ref[...], preferred_element_type=f32) \
            + be2_ref[...]
        att = 1.0 / (1.0 + jnp.exp(-logit))
        atts.append(att)
        agg = agg + att * Fsrc
    att_ref[0] = jnp.concatenate(atts, axis=1)

    ne = jnp.maximum(
        jnp.dot(nodes + agg, Wm_ref[...], preferred_element_type=f32)
        + bm_ref[...], 0.0)
    hidden = jax.nn.gelu(
        jnp.dot(ne, Wf1_ref[...], preferred_element_type=f32) + bf1_ref[...])
    nm = ne + jnp.dot(hidden, Wf2_ref[...], preferred_element_type=f32) \
        + bf2_ref[...]
    gv_ref[0, 0, :] = jnp.sum(nm, axis=0) * f32(1.0 / N)


def _head_body(g_ref, Wp1_ref, bp1_ref, gamma_ref, beta_ref, wp2r_ref,
               bp2_ref, out_ref):
    f32 = jnp.float32
    p = jnp.dot(g_ref[...], Wp1_ref[...], preferred_element_type=f32) \
        + bp1_ref[...]
    p = jax.nn.gelu(p * gamma_ref[...] + beta_ref[...])
    out_ref[...] = jnp.sum(p * wp2r_ref[...], axis=1, keepdims=True) \
        + bp2_ref[...]


def kernel(x, W_stem, b_stem, pos_embed, W_e1, b_e1, W_e2, b_e2, W_msg, b_msg,
           W_f1, b_f1, W_f2, b_f2, W_p1, b_p1, gamma, beta, W_p2, b_p2):
    f32 = jnp.float32
    patches = x.reshape(B, 3, H, P, W, P).transpose(0, 2, 4, 1, 3, 5) \
        .reshape(B, N, PATCH)
    Ws = W_stem.reshape(C, PATCH).T
    pos = pos_embed.transpose(0, 2, 3, 1).reshape(N, C)
    W1a = W_e1[:C]
    W1b = W_e1[C:]

    const2d = lambda: pl.BlockSpec(index_map=lambda i: (0, 0))
    att, gv = pl.pallas_call(
        _gsat_body,
        grid=(B,),
        in_specs=[
            pl.BlockSpec((1, N, PATCH), lambda i: (i, 0, 0)),
            const2d(),
            const2d(),
            const2d(),
            const2d(),
            const2d(),
            const2d(),
            const2d(),
            const2d(),
            const2d(),
            const2d(),
            const2d(),
            const2d(),
            const2d(),
            const2d(),
        ],
        out_specs=[
            pl.BlockSpec((1, N, K), lambda i: (i, 0, 0)),
            pl.BlockSpec((1, 1, C), lambda i: (i, 0, 0)),
        ],
        out_shape=[
            jax.ShapeDtypeStruct((B, N, K), f32),
            jax.ShapeDtypeStruct((B, 1, C), f32),
        ],
        compiler_params=pltpu.CompilerParams(
            dimension_semantics=("parallel",)),
    )(patches, pos, Ws, b_stem.reshape(1, C), W1a, W1b, b_e1.reshape(1, C),
      W_e2, b_e2.reshape(1, 1), W_msg, b_msg.reshape(1, C),
      W_f1, b_f1.reshape(1, 4 * C), W_f2, b_f2.reshape(1, C))

    pred = pl.pallas_call(
        _head_body,
        out_shape=jax.ShapeDtypeStruct((B, 1), f32),
    )(gv.reshape(B, C), W_p1, b_p1.reshape(1, 1024), gamma.reshape(1, 1024),
      beta.reshape(1, 1024), W_p2.reshape(1, 1024), b_p2.reshape(1, 1))

    return (att.reshape(B * N * K, 1), pred)


# speedup vs baseline: 3.9349x; 1.0356x over previous
import jax
import jax.numpy as jnp
from jax import lax
from jax.experimental import pallas as pl
from jax.experimental.pallas import tpu as pltpu

B = 64
C = 192
P = 16
H = 14
W = 14
N = H * W
K = 5
PATCH = 3 * P * P


IPS = 2


def _gsat_body(patches_ref, pos_ref, Ws_ref, bstem_ref, W1a_ref, W1b_ref,
               be1_ref, w2c_ref, be2_ref, Wm_ref, bm_ref, Wf1_ref, bf1_ref,
               Wf2_ref, bf2_ref, att_ref, gv_ref):
    f32 = jnp.float32
    ri = lax.broadcasted_iota(jnp.int32, (N, N), 0)
    ci = lax.broadcasted_iota(jnp.int32, (N, N), 1)
    for g in range(IPS):
        patches = patches_ref[g]
        nodes = jnp.dot(patches, Ws_ref[...], preferred_element_type=f32)
        nodes = nodes + bstem_ref[...] + pos_ref[...]
        A = jnp.dot(nodes, W1a_ref[...], preferred_element_type=f32)
        Bm = jnp.dot(nodes, W1b_ref[...], preferred_element_type=f32) \
            + be1_ref[...]

        G = lax.dot_general(nodes, nodes, (((1,), (1,)), ((), ())),
                            preferred_element_type=f32)
        nn = nodes * nodes
        sqrow = lax.dot_general(jnp.ones((1, C), f32), nn,
                                (((1,), (1,)), ((), ())),
                                preferred_element_type=f32,
                                precision=lax.Precision.HIGHEST)
        scores = sqrow - 2.0 * G
        scores = jnp.where(ri == ci, f32(1e10), scores)

        agg = jnp.zeros((N, C), f32)
        logits = []
        fsrcs = []
        for _ in range(K):
            m = jnp.min(scores, axis=1, keepdims=True)
            amin = jnp.min(jnp.where(scores == m, ci, N), axis=1,
                           keepdims=True)
            sel = ci == amin
            onehot = jnp.where(sel, f32(1.0), f32(0.0))
            scores = jnp.where(sel, f32(1e30), scores)
            Asrc = jnp.dot(onehot, A, preferred_element_type=f32)
            Fsrc = jnp.dot(onehot, nodes, preferred_element_type=f32)
            e = jnp.maximum(Asrc + Bm, 0.0)
            logits.append(jnp.dot(e, w2c_ref[...],
                                  preferred_element_type=f32))
            fsrcs.append(Fsrc)
        att = 1.0 / (1.0 + jnp.exp(-(jnp.concatenate(logits, axis=1)
                                     + be2_ref[...])))
        for k in range(K):
            agg = agg + lax.slice(att, (0, k), (N, k + 1)) * fsrcs[k]
        att_ref[g] = att

        ne = jnp.maximum(
            jnp.dot(nodes + agg, Wm_ref[...], preferred_element_type=f32)
            + bm_ref[...], 0.0)
        hidden = jax.nn.gelu(
            jnp.dot(ne, Wf1_ref[...], preferred_element_type=f32)
            + bf1_ref[...])
        nm = ne + jnp.dot(hidden, Wf2_ref[...], preferred_element_type=f32) \
            + bf2_ref[...]
        gv_ref[g, 0, :] = jnp.sum(nm, axis=0) * f32(1.0 / N)


def _head_body(g_ref, Wp1_ref, bp1_ref, gamma_ref, beta_ref, wp2r_ref,
               bp2_ref, out_ref):
    f32 = jnp.float32
    p = jnp.dot(g_ref[...], Wp1_ref[...], preferred_element_type=f32) \
        + bp1_ref[...]
    p = jax.nn.gelu(p * gamma_ref[...] + beta_ref[...])
    out_ref[...] = jnp.sum(p * wp2r_ref[...], axis=1, keepdims=True) \
        + bp2_ref[...]


def kernel(x, W_stem, b_stem, pos_embed, W_e1, b_e1, W_e2, b_e2, W_msg, b_msg,
           W_f1, b_f1, W_f2, b_f2, W_p1, b_p1, gamma, beta, W_p2, b_p2):
    f32 = jnp.float32
    patches = x.reshape(B, 3, H, P, W, P).transpose(0, 2, 4, 1, 3, 5) \
        .reshape(B, N, PATCH)
    Ws = W_stem.reshape(C, PATCH).T
    pos = pos_embed.transpose(0, 2, 3, 1).reshape(N, C)
    W1a = W_e1[:C]
    W1b = W_e1[C:]

    const2d = lambda: pl.BlockSpec(index_map=lambda i: (0, 0))
    att, gv = pl.pallas_call(
        _gsat_body,
        grid=(B // IPS,),
        in_specs=[
            pl.BlockSpec((IPS, N, PATCH), lambda i: (i, 0, 0)),
            const2d(),
            const2d(),
            const2d(),
            const2d(),
            const2d(),
            const2d(),
            const2d(),
            const2d(),
            const2d(),
            const2d(),
            const2d(),
            const2d(),
            const2d(),
            const2d(),
        ],
        out_specs=[
            pl.BlockSpec((IPS, N, K), lambda i: (i, 0, 0)),
            pl.BlockSpec((IPS, 1, C), lambda i: (i, 0, 0)),
        ],
        out_shape=[
            jax.ShapeDtypeStruct((B, N, K), f32),
            jax.ShapeDtypeStruct((B, 1, C), f32),
        ],
        compiler_params=pltpu.CompilerParams(
            dimension_semantics=("parallel",)),
    )(patches, pos, Ws, b_stem.reshape(1, C), W1a, W1b, b_e1.reshape(1, C),
      W_e2, b_e2.reshape(1, 1), W_msg, b_msg.reshape(1, C),
      W_f1, b_f1.reshape(1, 4 * C), W_f2, b_f2.reshape(1, C))

    pred = pl.pallas_call(
        _head_body,
        out_shape=jax.ShapeDtypeStruct((B, 1), f32),
    )(gv.reshape(B, C), W_p1, b_p1.reshape(1, 1024), gamma.reshape(1, 1024),
      beta.reshape(1, 1024), W_p2.reshape(1, 1024), b_p2.reshape(1, 1))

    return (att.reshape(B * N * K, 1), pred)
